# baseline (speedup 1.0000x reference)
import jax
import jax.numpy as jnp
from jax import lax
from jax.experimental import pallas as pl
from jax.experimental.pallas import tpu as pltpu
from jax.experimental.pallas import tpu_sc as plsc

B = 4096
L = 50
EMB = 64
LABEL = 30
NEG = 5

NC = 2
NS = 16
NW = NC * NS
BPW = B // NW
NLANE = 16
KSUB = EMB // NLANE


def _reduce_row(buf, acc, r):
  zero = jnp.zeros((NLANE,), jnp.float32)

  def body(j, carry):
    out = []
    for k in range(KSUB):
      a = carry[k]
      for l in range(2):
        a = a + buf[2 * j + l, pl.ds(k * NLANE, NLANE)]
      out.append(a)
    return tuple(out)

  accs = lax.fori_loop(0, L // 2, body, (zero,) * KSUB)
  for k in range(KSUB):
    acc[r, pl.ds(k * NLANE, NLANE)] = accs[k]


def _pool_body(x_hbm, table_hbm, out_hbm, idx_all, buf0, buf1, acc,
               sem0, sem1):
  wid = lax.axis_index("s") * NC + lax.axis_index("c")
  base = wid * BPW

  pltpu.sync_copy(x_hbm.at[pl.ds(base, BPW)], idx_all)

  def gather(r, buf, sem):
    return pltpu.make_async_copy(table_hbm.at[idx_all.at[r]], buf, sem)

  gather(0, buf0, sem0).start()

  def step(i, carry):
    r0 = 2 * i
    gather(r0 + 1, buf1, sem1).start()
    gather(r0, buf0, sem0).wait()
    _reduce_row(buf0, acc, r0)

    @pl.when(r0 + 2 < BPW)
    def _():
      gather(r0 + 2, buf0, sem0).start()

    gather(r0 + 1, buf1, sem1).wait()
    _reduce_row(buf1, acc, r0 + 1)
    return carry

  lax.fori_loop(0, BPW // 2, step, 0)

  pltpu.sync_copy(acc, out_hbm.at[pl.ds(base, BPW)])


@jax.jit
def _pool(x, table):
  mesh = plsc.VectorSubcoreMesh(
      core_axis_name="c", subcore_axis_name="s",
      num_cores=NC, num_subcores=NS)
  f = pl.kernel(
      _pool_body,
      out_type=jax.ShapeDtypeStruct((B, EMB), jnp.float32),
      mesh=mesh,
      compiler_params=pltpu.CompilerParams(use_tc_tiling_on_sc=False),
      scratch_types=[
          pltpu.VMEM((BPW, L), jnp.int32),
          pltpu.VMEM((L, EMB), jnp.float32),
          pltpu.VMEM((L, EMB), jnp.float32),
          pltpu.VMEM((BPW, EMB), jnp.float32),
          pltpu.SemaphoreType.DMA,
          pltpu.SemaphoreType.DMA,
      ],
  )
  return f(x, table)


def _dense_body(pooled_ref, mask_ref, y_ref, ob_ref, neg2_ref, w_ref,
                logit_ref, loss_ref):
  x_len = jnp.sum(mask_ref[...], axis=1, keepdims=True)
  user = pooled_ref[...] / x_len
  logit = lax.dot_general(user, w_ref[...],
                          (((1,), (1,)), ((), ())),
                          preferred_element_type=jnp.float32)
  logit_ref[...] = logit
  ob = ob_ref[...]
  wc = logit * ob
  yc = y_ref[...] * ob
  neg2 = neg2_ref[...]
  negsum = neg2[:, 0:LABEL]
  for n in range(1, NEG):
    negsum = negsum + neg2[:, n * LABEL:(n + 1) * LABEL]
  neg_term = jnp.log(jax.nn.sigmoid(-(negsum * wc)))
  total_neg = jnp.sum(neg_term)
  pos_in = jnp.sum(wc * yc, axis=1)
  pos_loss = jnp.sum(jnp.log(jax.nn.sigmoid(pos_in)))
  loss = -(LABEL * pos_loss + total_neg) / B
  loss_ref[...] = jnp.full((8, 128), loss, jnp.float32)


@jax.jit
def _dense(pooled, x_mask, y, ob, neg2, w):
  return pl.pallas_call(
      _dense_body,
      out_shape=[
          jax.ShapeDtypeStruct((B, LABEL), jnp.float32),
          jax.ShapeDtypeStruct((8, 128), jnp.float32),
      ],
  )(pooled, x_mask, y, ob, neg2, w)


def kernel(x, x_mask, y, ob, neg_samples, emb_table, W):
  pooled = _pool(x, emb_table)
  neg2 = jnp.reshape(neg_samples, (B, NEG * LABEL))
  logit, loss_tile = _dense(pooled, x_mask, y, ob, neg2, W)
  return logit, loss_tile[0, 0]
